# baseline (speedup 1.0000x reference)
import jax
import jax.numpy as jnp
from jax.experimental import pallas as pl
from jax.experimental.pallas import tpu as pltpu

_BATCH = 16384
_IN = 400
_OUT = 75
_GROUPS = 5
_GIN = 80
_GOUT = 15
_TM = 1024


def _mlp_block_kernel(x_ref, w_ref, b_ref, o_ref, wd_ref):
    wd_ref[...] = jnp.zeros((_IN, _OUT), dtype=jnp.float32)
    for i in range(_GROUPS):
        wd_ref[i * _GIN:(i + 1) * _GIN, i * _GOUT:(i + 1) * _GOUT] = (
            w_ref[i * _GOUT:(i + 1) * _GOUT, :].T)

    o_ref[...] = jax.lax.dot_general(
        x_ref[...], wd_ref[...],
        dimension_numbers=(((1,), (0,)), ((), ())),
        preferred_element_type=jnp.float32,
    ) + b_ref[...]


@jax.jit
def kernel(x, W0, W1, W2, W3, W4, b0, b1, b2, b3, b4):
    w = jnp.concatenate([W0, W1, W2, W3, W4], axis=0)
    b = jnp.concatenate([b0, b1, b2, b3, b4]).reshape(1, _OUT)
    grid = (_BATCH // _TM,)
    return pl.pallas_call(
        _mlp_block_kernel,
        grid=grid,
        in_specs=[
            pl.BlockSpec((_TM, _IN), lambda i: (i, 0)),
            pl.BlockSpec((_GROUPS * _GOUT, _GIN), lambda i: (0, 0)),
            pl.BlockSpec((1, _OUT), lambda i: (0, 0)),
        ],
        out_specs=pl.BlockSpec((_TM, _OUT), lambda i: (i, 0)),
        out_shape=jax.ShapeDtypeStruct((_BATCH, _OUT), jnp.float32),
        scratch_shapes=[pltpu.VMEM((_IN, _OUT), jnp.float32)],
        compiler_params=pltpu.CompilerParams(
            dimension_semantics=("parallel",),
        ),
    )(x, w, b)


# speedup vs baseline: 1.1044x; 1.1044x over previous
import jax
import jax.numpy as jnp
from jax.experimental import pallas as pl
from jax.experimental.pallas import tpu as pltpu

_BATCH = 16384
_IN = 400
_OUT = 75
_GROUPS = 5
_GIN = 80
_GOUT = 15
_TM = 2048


def _mlp_block_kernel(x_ref, w_ref, b_ref, o_ref, wd_ref):
    @pl.when(pl.program_id(0) == 0)
    def _build_block_diag():
        wd_ref[...] = jnp.zeros((_IN, _OUT), dtype=jnp.float32)
        for i in range(_GROUPS):
            wd_ref[i * _GIN:(i + 1) * _GIN, i * _GOUT:(i + 1) * _GOUT] = (
                w_ref[i * _GOUT:(i + 1) * _GOUT, :].T)

    o_ref[...] = jax.lax.dot_general(
        x_ref[...], wd_ref[...],
        dimension_numbers=(((1,), (0,)), ((), ())),
        preferred_element_type=jnp.float32,
    ) + b_ref[...]


@jax.jit
def kernel(x, W0, W1, W2, W3, W4, b0, b1, b2, b3, b4):
    w = jnp.concatenate([W0, W1, W2, W3, W4], axis=0)
    b = jnp.concatenate([b0, b1, b2, b3, b4]).reshape(1, _OUT)
    grid = (_BATCH // _TM,)
    return pl.pallas_call(
        _mlp_block_kernel,
        grid=grid,
        in_specs=[
            pl.BlockSpec((_TM, _IN), lambda i: (i, 0)),
            pl.BlockSpec((_GROUPS * _GOUT, _GIN), lambda i: (0, 0)),
            pl.BlockSpec((1, _OUT), lambda i: (0, 0)),
        ],
        out_specs=pl.BlockSpec((_TM, _OUT), lambda i: (i, 0)),
        out_shape=jax.ShapeDtypeStruct((_BATCH, _OUT), jnp.float32),
        scratch_shapes=[pltpu.VMEM((_IN, _OUT), jnp.float32)],
        compiler_params=pltpu.CompilerParams(
            dimension_semantics=("arbitrary",),
        ),
    )(x, w, b)


# speedup vs baseline: 1.1225x; 1.0164x over previous
import jax
import jax.numpy as jnp
from jax.experimental import pallas as pl
from jax.experimental.pallas import tpu as pltpu

_BATCH = 16384
_IN = 400
_OUT = 75
_GROUPS = 5
_GIN = 80
_GOUT = 15
_TM = 4096


def _mlp_block_kernel(x_ref, w_ref, b_ref, o_ref, wd_ref):
    @pl.when(pl.program_id(0) == 0)
    def _build_block_diag():
        wd_ref[...] = jnp.zeros((_IN, _OUT), dtype=jnp.float32)
        for i in range(_GROUPS):
            wd_ref[i * _GIN:(i + 1) * _GIN, i * _GOUT:(i + 1) * _GOUT] = (
                w_ref[i * _GOUT:(i + 1) * _GOUT, :].T)

    o_ref[...] = jax.lax.dot_general(
        x_ref[...], wd_ref[...],
        dimension_numbers=(((1,), (0,)), ((), ())),
        preferred_element_type=jnp.float32,
    ) + b_ref[...]


@jax.jit
def kernel(x, W0, W1, W2, W3, W4, b0, b1, b2, b3, b4):
    w = jnp.concatenate([W0, W1, W2, W3, W4], axis=0)
    b = jnp.concatenate([b0, b1, b2, b3, b4]).reshape(1, _OUT)
    grid = (_BATCH // _TM,)
    return pl.pallas_call(
        _mlp_block_kernel,
        grid=grid,
        in_specs=[
            pl.BlockSpec((_TM, _IN), lambda i: (i, 0)),
            pl.BlockSpec((_GROUPS * _GOUT, _GIN), lambda i: (0, 0)),
            pl.BlockSpec((1, _OUT), lambda i: (0, 0)),
        ],
        out_specs=pl.BlockSpec((_TM, _OUT), lambda i: (i, 0)),
        out_shape=jax.ShapeDtypeStruct((_BATCH, _OUT), jnp.float32),
        scratch_shapes=[pltpu.VMEM((_IN, _OUT), jnp.float32)],
        compiler_params=pltpu.CompilerParams(
            dimension_semantics=("arbitrary",),
        ),
    )(x, w, b)
